# baseline (speedup 1.0000x reference)
import jax
import jax.numpy as jnp
from jax import lax
from jax.experimental import pallas as pl
from jax.experimental.pallas import tpu as pltpu
from jax.experimental.pallas import tpu_sc as plsc

N = 128 * 128
K0 = N // 2
NCORES = 1
NT = 16
CHUNK = N // NT
NV = CHUNK // 16
LEVELS = [(22, 8), (14, 8), (6, 8), (0, 6)]


def _bf16_round(v):
    u = plsc.bitcast(v, jnp.int32)
    r = u + jnp.int32(0x7FFF) + ((u >> 16) & 1)
    return plsc.bitcast(r & jnp.int32(-0x10000), jnp.float32)


def _sc_body(x_hbm, w_hbm, newx_hbm, score_hbm, xv, wv, score_v, p_v,
             out_v, hist_v, hist_all_v, merged_v, tie_v, ties_all_v, hist_sh,
             ties_sh):
    cid = lax.axis_index("c")
    sid = lax.axis_index("s")
    base = CHUNK * sid
    lanes = jnp.arange(16, dtype=jnp.int32)

    xv[pl.ds(0, 16)] = jnp.zeros((16,), jnp.float32)
    xv[pl.ds(CHUNK + 16, 16)] = jnp.zeros((16,), jnp.float32)
    pltpu.sync_copy(x_hbm.at[pl.ds(base, CHUNK)], xv.at[pl.ds(16, CHUNK)])

    @pl.when(sid > 0)
    def _():
        pltpu.sync_copy(x_hbm.at[pl.ds(base - 16, 16)], xv.at[pl.ds(0, 16)])

    @pl.when(sid < NT - 1)
    def _():
        pltpu.sync_copy(
            x_hbm.at[pl.ds(base + CHUNK, 16)], xv.at[pl.ds(CHUNK + 16, 16)]
        )

    pltpu.sync_copy(w_hbm, wv)
    wvec = _bf16_round(wv[pl.ds(0, 16)])
    w = [wvec[d] for d in range(7)]

    def stage1(j, _):
        o = 16 * j
        t = []
        for d in range(7):
            xb = _bf16_round(xv[pl.ds(o + d + 13, 16)])
            t.append(xb * w[d])
        y = (((t[0] + t[1]) + (t[2] + t[3])) + (t[4] + t[5])) + t[6]
        s = 1.0 / (1.0 + jnp.exp(-y))
        score_v[pl.ds(o, 16)] = s
        p_v[pl.ds(o, 16)] = xv[pl.ds(o + 16, 16)] * (s + 1.0)
        return 0

    lax.fori_loop(0, NV, stage1, 0, unroll=4)
    half = CHUNK // NCORES
    pltpu.sync_copy(
        score_v.at[pl.ds(half * cid, half)],
        score_hbm.at[pl.ds(base + half * cid, half)],
    )

    rem = jnp.int32(K0)
    prefix = jnp.int32(0)
    ones = jnp.ones((16,), jnp.int32)
    for lvl, (shift, width) in enumerate(LEVELS):
        nb = 1 << width
        ngroups = nb // 16

        def zero_hist(g, _):
            hist_v[pl.ds(16 * g, 16)] = jnp.zeros((16,), jnp.int32)
            return 0

        lax.fori_loop(0, ngroups, zero_hist, 0)

        def build(j, _, shift=shift, width=width, prefix=prefix, nb=nb):
            key = plsc.bitcast(score_v[pl.ds(16 * j, 16)], jnp.int32)
            mask = (key >> (shift + width)) == prefix
            b = (key >> shift) & (nb - 1)
            plsc.addupdate_scatter(hist_v, [b], ones, mask=mask)
            return 0

        lax.fori_loop(0, NV, build, 0)
        pltpu.sync_copy(
            hist_v.at[pl.ds(0, nb)], hist_sh.at[lvl].at[sid].at[pl.ds(0, nb)]
        )
        plsc.subcore_barrier()
        pltpu.sync_copy(hist_sh.at[lvl], hist_all_v)

        def merge(g, _):
            acc = jnp.zeros((16,), jnp.int32)
            for r in range(NT):
                acc = acc + hist_all_v[r, pl.ds(16 * g, 16)]
            merged_v[pl.ds(16 * g, 16)] = acc
            return 0

        lax.fori_loop(0, ngroups, merge, 0)

        def scan(g, carry):
            found, b_sel, cb_sel, before = carry
            hv = merged_v[pl.ds(16 * g, 16)]
            c = plsc.cumsum(hv)
            tot = jnp.max(c)
            ge = (before + c) >= rem
            s = jnp.sum(ge.astype(jnp.int32))
            lane = 16 - s
            hit = jnp.logical_and(jnp.logical_not(found), s > 0)
            e = jnp.sum(jnp.where(lanes == lane, c - hv, 0))
            b_sel = jnp.where(hit, 16 * g + lane, b_sel)
            cb_sel = jnp.where(hit, before + e, cb_sel)
            found = jnp.logical_or(found, s > 0)
            return found, b_sel, cb_sel, before + tot

        init = (jnp.bool_(False), jnp.int32(0), jnp.int32(0), jnp.int32(0))
        _, b_sel, cb_sel, _ = lax.fori_loop(0, ngroups, scan, init)
        rem = rem - cb_sel
        prefix = (prefix << width) | b_sel

    T = prefix
    m = rem

    def tie_count(j, acc):
        key = plsc.bitcast(score_v[pl.ds(16 * j, 16)], jnp.int32)
        return acc + jnp.sum((key == T).astype(jnp.int32))

    my_ties = lax.fori_loop(0, NV, tie_count, jnp.int32(0))
    tie_v[pl.ds(0, 16)] = jnp.where(lanes == sid, my_ties, 0)
    pltpu.sync_copy(tie_v, ties_sh.at[cid].at[sid])
    plsc.subcore_barrier()
    pltpu.sync_copy(ties_sh.at[cid], ties_all_v)

    tie_all = jnp.zeros((16,), jnp.int32)
    for t in range(NT):
        tie_all = tie_all + ties_all_v[t, pl.ds(0, 16)]
    before_me = jnp.sum(jnp.where(lanes < sid, tie_all, 0))

    def emit(j, r):
        key = plsc.bitcast(score_v[pl.ds(16 * j, 16)], jnp.int32)
        tie = key == T
        inc = plsc.cumsum(tie.astype(jnp.int32))
        sel = jnp.logical_or(key < T, jnp.logical_and(tie, (r + inc) <= m))
        out_v[pl.ds(16 * j, 16)] = jnp.where(sel, p_v[pl.ds(16 * j, 16)], 0.0)
        return r + jnp.max(inc)

    lax.fori_loop(0, NV, emit, before_me)
    pltpu.sync_copy(
        out_v.at[pl.ds(half * cid, half)],
        newx_hbm.at[pl.ds(base + half * cid, half)],
    )


def kernel(x, conv_w):
    wb = jnp.pad(conv_w.reshape(7), (0, 9))
    mesh = plsc.VectorSubcoreMesh(
        core_axis_name="c", subcore_axis_name="s", num_cores=NCORES, num_subcores=16
    )
    newx, score = pl.kernel(
        _sc_body,
        out_type=(
            jax.ShapeDtypeStruct((N,), jnp.float32),
            jax.ShapeDtypeStruct((N,), jnp.float32),
        ),
        mesh=mesh,
        compiler_params=pltpu.CompilerParams(needs_layout_passes=False),
        scratch_types=[
            pltpu.VMEM((CHUNK + 32,), jnp.float32),
            pltpu.VMEM((16,), jnp.float32),
            pltpu.VMEM((CHUNK,), jnp.float32),
            pltpu.VMEM((CHUNK,), jnp.float32),
            pltpu.VMEM((CHUNK,), jnp.float32),
            pltpu.VMEM((256,), jnp.int32),
            pltpu.VMEM((NT, 256), jnp.int32),
            pltpu.VMEM((256,), jnp.int32),
            pltpu.VMEM((16,), jnp.int32),
            pltpu.VMEM((NT, 16), jnp.int32),
            pltpu.VMEM_SHARED((4, NT, 256), jnp.int32),
            pltpu.VMEM_SHARED((NCORES, NT, 16), jnp.int32),
        ],
    )(x, wb)
    return newx, score


# speedup vs baseline: 1.0528x; 1.0528x over previous
import functools

import jax
import jax.numpy as jnp
from jax import lax
from jax.experimental import pallas as pl
from jax.experimental.pallas import tpu as pltpu
from jax.experimental.pallas import tpu_sc as plsc

N = 128 * 128
K0 = N // 2
NCORES = 1
NT = 16
CHUNK = N // NT
NV = CHUNK // 16
LEVELS = [(22, 8), (14, 8), (6, 8), (0, 6)]


def _bf16_round(v):
    u = plsc.bitcast(v, jnp.int32)
    r = u + jnp.int32(0x7FFF) + ((u >> 16) & 1)
    return plsc.bitcast(r & jnp.int32(-0x10000), jnp.float32)


def _sc_body(xpad_hbm, w_hbm, newx_hbm, score_hbm, xv, wv, score_v, p_v,
             out_v, hist_v, hist_all_v, merged_v, tie_v, ties_all_v, hist_sh,
             ties_sh):
    cid = lax.axis_index("c")
    sid = lax.axis_index("s")
    base = CHUNK * sid
    lanes = jnp.arange(16, dtype=jnp.int32)

    pltpu.sync_copy(xpad_hbm.at[pl.ds(base, CHUNK + 8)], xv)
    pltpu.sync_copy(w_hbm, wv)
    wvec = wv[pl.ds(0, 16)]
    w = [wvec[d] for d in range(7)]

    def stage1(j, _):
        o = 16 * j
        t = []
        for d in range(7):
            xb = _bf16_round(xv[pl.ds(o + d, 16)])
            t.append(xb * w[d])
        y = (((t[0] + t[1]) + (t[2] + t[3])) + (t[4] + t[5])) + t[6]
        s = 1.0 / (1.0 + jnp.exp(-y))
        score_v[pl.ds(o, 16)] = s
        p_v[pl.ds(o, 16)] = xv[pl.ds(o + 3, 16)] * (s + 1.0)
        return 0

    lax.fori_loop(0, NV, stage1, 0)
    half = CHUNK // NCORES
    pltpu.sync_copy(
        score_v.at[pl.ds(half * cid, half)],
        score_hbm.at[pl.ds(base + half * cid, half)],
    )

    rem = jnp.int32(K0)
    prefix = jnp.int32(0)
    ones = jnp.ones((16,), jnp.int32)
    for lvl, (shift, width) in enumerate(LEVELS):
        nb = 1 << width
        ngroups = nb // 16

        def zero_hist(g, _):
            hist_v[pl.ds(16 * g, 16)] = jnp.zeros((16,), jnp.int32)
            return 0

        lax.fori_loop(0, ngroups, zero_hist, 0)

        def build(j, _, shift=shift, width=width, prefix=prefix, nb=nb):
            key = plsc.bitcast(score_v[pl.ds(16 * j, 16)], jnp.int32)
            mask = (key >> (shift + width)) == prefix
            b = (key >> shift) & (nb - 1)
            plsc.addupdate_scatter(hist_v, [b], ones, mask=mask)
            return 0

        lax.fori_loop(0, NV, build, 0)
        pltpu.sync_copy(
            hist_v.at[pl.ds(0, nb)], hist_sh.at[lvl].at[sid].at[pl.ds(0, nb)]
        )
        plsc.subcore_barrier()
        pltpu.sync_copy(hist_sh.at[lvl], hist_all_v)

        def merge(g, _):
            acc = jnp.zeros((16,), jnp.int32)
            for r in range(NT):
                acc = acc + hist_all_v[r, pl.ds(16 * g, 16)]
            merged_v[pl.ds(16 * g, 16)] = acc
            return 0

        lax.fori_loop(0, ngroups, merge, 0)

        def scan(g, carry):
            found, b_sel, cb_sel, before = carry
            hv = merged_v[pl.ds(16 * g, 16)]
            c = plsc.cumsum(hv)
            tot = jnp.max(c)
            ge = (before + c) >= rem
            s = jnp.sum(ge.astype(jnp.int32))
            lane = 16 - s
            hit = jnp.logical_and(jnp.logical_not(found), s > 0)
            e = jnp.sum(jnp.where(lanes == lane, c - hv, 0))
            b_sel = jnp.where(hit, 16 * g + lane, b_sel)
            cb_sel = jnp.where(hit, before + e, cb_sel)
            found = jnp.logical_or(found, s > 0)
            return found, b_sel, cb_sel, before + tot

        init = (jnp.bool_(False), jnp.int32(0), jnp.int32(0), jnp.int32(0))
        _, b_sel, cb_sel, _ = lax.fori_loop(0, ngroups, scan, init)
        rem = rem - cb_sel
        prefix = (prefix << width) | b_sel

    T = prefix
    m = rem

    def tie_count(j, acc):
        key = plsc.bitcast(score_v[pl.ds(16 * j, 16)], jnp.int32)
        return acc + jnp.sum((key == T).astype(jnp.int32))

    my_ties = lax.fori_loop(0, NV, tie_count, jnp.int32(0))
    tie_v[pl.ds(0, 16)] = jnp.where(lanes == sid, my_ties, 0)
    pltpu.sync_copy(tie_v, ties_sh.at[cid].at[sid])
    plsc.subcore_barrier()
    pltpu.sync_copy(ties_sh.at[cid], ties_all_v)

    tie_all = jnp.zeros((16,), jnp.int32)
    for t in range(NT):
        tie_all = tie_all + ties_all_v[t, pl.ds(0, 16)]
    before_me = jnp.sum(jnp.where(lanes < sid, tie_all, 0))

    def emit(j, r):
        key = plsc.bitcast(score_v[pl.ds(16 * j, 16)], jnp.int32)
        tie = key == T
        inc = plsc.cumsum(tie.astype(jnp.int32))
        sel = jnp.logical_or(key < T, jnp.logical_and(tie, (r + inc) <= m))
        out_v[pl.ds(16 * j, 16)] = jnp.where(sel, p_v[pl.ds(16 * j, 16)], 0.0)
        return r + jnp.max(inc)

    lax.fori_loop(0, NV, emit, before_me)
    pltpu.sync_copy(
        out_v.at[pl.ds(half * cid, half)],
        newx_hbm.at[pl.ds(base + half * cid, half)],
    )


def kernel(x, conv_w):
    xpad = jnp.pad(x, (3, 5))
    wb = conv_w.reshape(7).astype(jnp.bfloat16).astype(jnp.float32)
    wb = jnp.pad(wb, (0, 9))
    mesh = plsc.VectorSubcoreMesh(
        core_axis_name="c", subcore_axis_name="s", num_cores=NCORES, num_subcores=16
    )
    newx, score = pl.kernel(
        _sc_body,
        out_type=(
            jax.ShapeDtypeStruct((N,), jnp.float32),
            jax.ShapeDtypeStruct((N,), jnp.float32),
        ),
        mesh=mesh,
        compiler_params=pltpu.CompilerParams(needs_layout_passes=False),
        scratch_types=[
            pltpu.VMEM((CHUNK + 8,), jnp.float32),
            pltpu.VMEM((16,), jnp.float32),
            pltpu.VMEM((CHUNK,), jnp.float32),
            pltpu.VMEM((CHUNK,), jnp.float32),
            pltpu.VMEM((CHUNK,), jnp.float32),
            pltpu.VMEM((256,), jnp.int32),
            pltpu.VMEM((NT, 256), jnp.int32),
            pltpu.VMEM((256,), jnp.int32),
            pltpu.VMEM((16,), jnp.int32),
            pltpu.VMEM((NT, 16), jnp.int32),
            pltpu.VMEM_SHARED((4, NT, 256), jnp.int32),
            pltpu.VMEM_SHARED((NCORES, NT, 16), jnp.int32),
        ],
    )(xpad, wb)
    return newx, score


# speedup vs baseline: 1.1014x; 1.0462x over previous
import jax
import jax.numpy as jnp
from jax import lax
from jax.experimental import pallas as pl
from jax.experimental.pallas import tpu as pltpu
from jax.experimental.pallas import tpu_sc as plsc

N = 128 * 128
K0 = N // 2
NT = 16
CHUNK = N // NT
NV = CHUNK // 16
R, C = 128, 128
LEVELS = [(22, 8), (14, 8), (6, 8), (0, 6)]


def _tc_body(w_ref, x0, x1, x2, x3, x4, x5, x6, score_ref, p_ref):
    xs = (x0, x1, x2, x3, x4, x5, x6)
    xb = [xs[d][...].astype(jnp.bfloat16).astype(jnp.float32) for d in range(7)]
    wb = [w_ref[d].astype(jnp.bfloat16).astype(jnp.float32) for d in range(7)]
    t = [xb[d] * wb[d] for d in range(7)]
    y = (((t[0] + t[1]) + (t[2] + t[3])) + (t[4] + t[5])) + t[6]
    score = jax.nn.sigmoid(y)
    score_ref[...] = score
    p_ref[...] = x3[...] * (score + 1.0)


def _sc_body(score_hbm, p_hbm, newx_hbm, score_v, p_v, out_v, hist_v,
             hist_all_v, merged_v, tie_v, ties_all_v, hist_sh, ties_sh):
    sid = lax.axis_index("s")
    base = CHUNK * sid
    lanes = jnp.arange(16, dtype=jnp.int32)

    pltpu.sync_copy(score_hbm.at[pl.ds(base, CHUNK)], score_v)
    pltpu.sync_copy(p_hbm.at[pl.ds(base, CHUNK)], p_v)

    rem = jnp.int32(K0)
    prefix = jnp.int32(0)
    ones = jnp.ones((16,), jnp.int32)
    for lvl, (shift, width) in enumerate(LEVELS):
        nb = 1 << width
        ngroups = nb // 16

        def zero_hist(g, _):
            hist_v[pl.ds(16 * g, 16)] = jnp.zeros((16,), jnp.int32)
            return 0

        lax.fori_loop(0, ngroups, zero_hist, 0)

        def build(j, _, shift=shift, width=width, prefix=prefix, nb=nb):
            key = plsc.bitcast(score_v[pl.ds(16 * j, 16)], jnp.int32)
            mask = (key >> (shift + width)) == prefix
            b = (key >> shift) & (nb - 1)
            plsc.addupdate_scatter(hist_v, [b], ones, mask=mask)
            return 0

        lax.fori_loop(0, NV, build, 0)
        pltpu.sync_copy(
            hist_v.at[pl.ds(0, nb)], hist_sh.at[lvl].at[sid].at[pl.ds(0, nb)]
        )
        plsc.subcore_barrier()
        pltpu.sync_copy(hist_sh.at[lvl], hist_all_v)

        def merge(g, _):
            acc = jnp.zeros((16,), jnp.int32)
            for r in range(NT):
                acc = acc + hist_all_v[r, pl.ds(16 * g, 16)]
            merged_v[pl.ds(16 * g, 16)] = acc
            return 0

        lax.fori_loop(0, ngroups, merge, 0)

        def scan(g, carry):
            found, b_sel, cb_sel, before = carry
            hv = merged_v[pl.ds(16 * g, 16)]
            c = plsc.cumsum(hv)
            tot = jnp.max(c)
            ge = (before + c) >= rem
            s = jnp.sum(ge.astype(jnp.int32))
            lane = 16 - s
            hit = jnp.logical_and(jnp.logical_not(found), s > 0)
            e = jnp.sum(jnp.where(lanes == lane, c - hv, 0))
            b_sel = jnp.where(hit, 16 * g + lane, b_sel)
            cb_sel = jnp.where(hit, before + e, cb_sel)
            found = jnp.logical_or(found, s > 0)
            return found, b_sel, cb_sel, before + tot

        init = (jnp.bool_(False), jnp.int32(0), jnp.int32(0), jnp.int32(0))
        _, b_sel, cb_sel, _ = lax.fori_loop(0, ngroups, scan, init)
        rem = rem - cb_sel
        prefix = (prefix << width) | b_sel

    T = prefix
    m = rem

    def tie_count(j, acc):
        key = plsc.bitcast(score_v[pl.ds(16 * j, 16)], jnp.int32)
        return acc + jnp.sum((key == T).astype(jnp.int32))

    my_ties = lax.fori_loop(0, NV, tie_count, jnp.int32(0))
    tie_v[pl.ds(0, 16)] = jnp.where(lanes == sid, my_ties, 0)
    pltpu.sync_copy(tie_v, ties_sh.at[sid])
    plsc.subcore_barrier()
    pltpu.sync_copy(ties_sh, ties_all_v)

    tie_all = jnp.zeros((16,), jnp.int32)
    for t in range(NT):
        tie_all = tie_all + ties_all_v[t, pl.ds(0, 16)]
    before_me = jnp.sum(jnp.where(lanes < sid, tie_all, 0))

    def emit(j, r):
        key = plsc.bitcast(score_v[pl.ds(16 * j, 16)], jnp.int32)
        tie = key == T
        inc = plsc.cumsum(tie.astype(jnp.int32))
        sel = jnp.logical_or(key < T, jnp.logical_and(tie, (r + inc) <= m))
        out_v[pl.ds(16 * j, 16)] = jnp.where(sel, p_v[pl.ds(16 * j, 16)], 0.0)
        return r + jnp.max(inc)

    lax.fori_loop(0, NV, emit, before_me)
    pltpu.sync_copy(out_v, newx_hbm.at[pl.ds(base, CHUNK)])


def kernel(x, conv_w):
    xp = jnp.pad(x, (3, 3))
    xs = [xp[d : d + N].reshape(R, C) for d in range(7)]
    w = conv_w.reshape(7)
    score2d, p2d = pl.pallas_call(
        _tc_body,
        out_shape=(
            jax.ShapeDtypeStruct((R, C), jnp.float32),
            jax.ShapeDtypeStruct((R, C), jnp.float32),
        ),
        in_specs=[pl.BlockSpec(memory_space=pltpu.SMEM)]
        + [pl.BlockSpec(memory_space=pltpu.VMEM)] * 7,
        out_specs=(
            pl.BlockSpec(memory_space=pltpu.VMEM),
            pl.BlockSpec(memory_space=pltpu.VMEM),
        ),
    )(w, *xs)
    score = score2d.reshape(N)
    p = p2d.reshape(N)

    mesh = plsc.VectorSubcoreMesh(
        core_axis_name="c", subcore_axis_name="s", num_cores=1, num_subcores=16
    )
    newx = pl.kernel(
        _sc_body,
        out_type=jax.ShapeDtypeStruct((N,), jnp.float32),
        mesh=mesh,
        compiler_params=pltpu.CompilerParams(needs_layout_passes=False),
        scratch_types=[
            pltpu.VMEM((CHUNK,), jnp.float32),
            pltpu.VMEM((CHUNK,), jnp.float32),
            pltpu.VMEM((CHUNK,), jnp.float32),
            pltpu.VMEM((256,), jnp.int32),
            pltpu.VMEM((NT, 256), jnp.int32),
            pltpu.VMEM((256,), jnp.int32),
            pltpu.VMEM((16,), jnp.int32),
            pltpu.VMEM((NT, 16), jnp.int32),
            pltpu.VMEM_SHARED((4, NT, 256), jnp.int32),
            pltpu.VMEM_SHARED((NT, 16), jnp.int32),
        ],
    )(score, p)
    return newx, score
